# baseline (speedup 1.0000x reference)
import functools

import jax
import jax.numpy as jnp
from jax import lax
from jax.experimental import pallas as pl
from jax.experimental.pallas import tpu as pltpu
from jax.experimental.pallas import tpu_sc as plsc

N = 10000
E = 320000
D_FEAT = 128
D_EDGE = 16
NTYPE_DIM = 64
ETYPE_DIM = 16

NC = 2
NS = 16
NW = NC * NS

NODE_PW = 312
NODE_TAIL = N - NW * NODE_PW
NODE_G = 104

EDGE_PW = E // NW
EC = 1000
NB = EDGE_PW // EC
EDGE_SUBS = [(i * 128, 128) for i in range(7)] + [(896, 104)]


def _body(x, eattr, ntypes, etypes, ntab, etab, xcat, ecat,
          nidx_v, nemb_v, nx_v, eidx_v, eemb_v, efeat_v,
          s_idx, s_feat, s_g, s_oemb, s_ofeat):
    wid = lax.axis_index("s") * NC + lax.axis_index("c")

    nbase = pl.multiple_of(wid * NODE_PW, 8)
    c_idx = pltpu.async_copy(ntypes.at[pl.ds(nbase, NODE_PW)], nidx_v, s_idx)
    c_x = pltpu.async_copy(x.at[pl.ds(nbase, NODE_PW)], nx_v, s_feat)
    c_idx.wait()
    gathers = [
        pltpu.async_copy(ntab.at[nidx_v.at[pl.ds(off, sz)]],
                         nemb_v.at[pl.ds(off, sz)], s_g)
        for off, sz in [(0, NODE_G), (NODE_G, NODE_G), (2 * NODE_G, NODE_G)]
    ]
    c_x.wait()
    c_ox = pltpu.async_copy(
        nx_v, xcat.at[pl.ds(nbase, NODE_PW), pl.ds(NTYPE_DIM, D_FEAT)], s_ofeat)
    for g in gathers:
        g.wait()
    c_oe = pltpu.async_copy(
        nemb_v, xcat.at[pl.ds(nbase, NODE_PW), pl.ds(0, NTYPE_DIM)], s_oemb)
    c_ox.wait()
    c_oe.wait()

    @pl.when(wid == 0)
    def _node_tail():
        tbase = NW * NODE_PW
        pltpu.sync_copy(ntypes.at[pl.ds(tbase, NODE_TAIL)],
                        nidx_v.at[pl.ds(0, NODE_TAIL)])
        pltpu.async_copy(ntab.at[nidx_v.at[pl.ds(0, NODE_TAIL)]],
                         nemb_v.at[pl.ds(0, NODE_TAIL)], s_g).wait()
        pltpu.sync_copy(x.at[pl.ds(tbase, NODE_TAIL)],
                        nx_v.at[pl.ds(0, NODE_TAIL)])
        pltpu.sync_copy(nemb_v.at[pl.ds(0, NODE_TAIL)],
                        xcat.at[pl.ds(tbase, NODE_TAIL), pl.ds(0, NTYPE_DIM)])
        pltpu.sync_copy(nx_v.at[pl.ds(0, NODE_TAIL)],
                        xcat.at[pl.ds(tbase, NODE_TAIL), pl.ds(NTYPE_DIM, D_FEAT)])

    ebase = wid * EDGE_PW

    def edge_chunk(k, carry):
        base = pl.multiple_of(ebase + k * EC, 8)
        c_idx = pltpu.async_copy(etypes.at[pl.ds(base, EC)], eidx_v, s_idx)
        c_feat = pltpu.async_copy(eattr.at[pl.ds(base, EC)], efeat_v, s_feat)
        c_idx.wait()
        gathers = [
            pltpu.async_copy(etab.at[eidx_v.at[pl.ds(off, sz)]],
                             eemb_v.at[pl.ds(off, sz)], s_g)
            for off, sz in EDGE_SUBS
        ]
        c_feat.wait()
        c_of = pltpu.async_copy(
            efeat_v, ecat.at[pl.ds(base, EC), pl.ds(ETYPE_DIM, D_EDGE)], s_ofeat)
        for g in gathers:
            g.wait()
        c_oe = pltpu.async_copy(
            eemb_v, ecat.at[pl.ds(base, EC), pl.ds(0, ETYPE_DIM)], s_oemb)
        c_of.wait()
        c_oe.wait()
        return carry

    lax.fori_loop(0, NB, edge_chunk, 0)


@functools.partial(jax.jit, static_argnames=())
def kernel(x, eattr, ntypes, etypes, ntype_table, etype_table):
    run = pl.kernel(
        _body,
        out_type=(
            jax.ShapeDtypeStruct((N, NTYPE_DIM + D_FEAT), jnp.float32),
            jax.ShapeDtypeStruct((E, ETYPE_DIM + D_EDGE), jnp.float32),
        ),
        mesh=plsc.VectorSubcoreMesh(core_axis_name="c", subcore_axis_name="s"),
        compiler_params=pltpu.CompilerParams(use_tc_tiling_on_sc=False),
        scratch_types=[
            pltpu.VMEM((NODE_PW,), jnp.int32),
            pltpu.VMEM((NODE_PW, NTYPE_DIM), jnp.float32),
            pltpu.VMEM((NODE_PW, D_FEAT), jnp.float32),
            pltpu.VMEM((EC,), jnp.int32),
            pltpu.VMEM((EC, ETYPE_DIM), jnp.float32),
            pltpu.VMEM((EC, D_EDGE), jnp.float32),
            pltpu.SemaphoreType.DMA,
            pltpu.SemaphoreType.DMA,
            pltpu.SemaphoreType.DMA,
            pltpu.SemaphoreType.DMA,
            pltpu.SemaphoreType.DMA,
        ],
    )
    x_cat, eattr_cat = run(x, eattr, ntypes.astype(jnp.int32),
                           etypes.astype(jnp.int32), ntype_table, etype_table)
    return (x_cat, eattr_cat)


# speedup vs baseline: 1.6263x; 1.6263x over previous
import functools

import jax
import jax.numpy as jnp
from jax import lax
from jax.experimental import pallas as pl
from jax.experimental.pallas import tpu as pltpu
from jax.experimental.pallas import tpu_sc as plsc

N = 10000
E = 320000
D_FEAT = 128
D_EDGE = 16
NTYPE_DIM = 64
ETYPE_DIM = 16
NODE_W = NTYPE_DIM + D_FEAT
EDGE_W = ETYPE_DIM + D_EDGE
NUM_NTYPES = 119
NUM_ETYPES = 22

NC = 2
NS = 16
NW = NC * NS
L = 16

NCH = 80
N_CHUNKS = N // NCH
NODE_ITERS = -(-N_CHUNKS // NW)

EDGE_PW = E // NW
EC = 2000
NB = EDGE_PW // EC


def _body(x, eattr, ntypes, etypes, ntab, etab, xcat, ecat,
          ntab_v, etab_v, nidx_v, nrow_v, eidx_v, erow_v,
          s_idx, s_feat, s_out):
    wid = lax.axis_index("s") * NC + lax.axis_index("c")
    iota = lax.broadcasted_iota(jnp.int32, (L,), 0)

    pltpu.sync_copy(ntab, ntab_v)
    pltpu.sync_copy(etab, etab_v)

    ebase = wid * EDGE_PW

    def edge_chunk(k, carry):
        base = pl.multiple_of(ebase + k * EC, 8)
        c_idx = pltpu.async_copy(etypes.at[pl.ds(base, EC)], eidx_v, s_idx)
        c_feat = pltpu.async_copy(eattr.at[pl.ds(base, EC)],
                                  erow_v.at[:, pl.ds(ETYPE_DIM, D_EDGE)], s_feat)
        c_idx.wait()

        def group(g, carry2):
            ev = eidx_v[pl.ds(g * L, L)]
            rowv = iota + g * L
            for d in range(ETYPE_DIM):
                dv = jnp.full((L,), d, jnp.int32)
                vals = plsc.load_gather(etab_v, [ev, dv])
                plsc.store_scatter(erow_v, [rowv, dv], vals)
            return carry2

        lax.fori_loop(0, EC // L, group, 0)
        c_feat.wait()
        pltpu.async_copy(erow_v, ecat.at[pl.ds(base, EC)], s_out).wait()
        return carry

    lax.fori_loop(0, NB, edge_chunk, 0)

    def node_iter(k, carry):
        c = wid + k * NW

        @pl.when(c < N_CHUNKS)
        def _():
            base = pl.multiple_of(c * NCH, 8)
            c_idx = pltpu.async_copy(ntypes.at[pl.ds(base, NCH)], nidx_v, s_idx)
            c_x = pltpu.async_copy(x.at[pl.ds(base, NCH)],
                                   nrow_v.at[:, pl.ds(NTYPE_DIM, D_FEAT)], s_feat)
            c_idx.wait()

            def group(g, carry2):
                nv = nidx_v[pl.ds(g * L, L)]
                rowv = iota + g * L
                for d in range(NTYPE_DIM):
                    dv = jnp.full((L,), d, jnp.int32)
                    vals = plsc.load_gather(ntab_v, [nv, dv])
                    plsc.store_scatter(nrow_v, [rowv, dv], vals)
                return carry2

            lax.fori_loop(0, NCH // L, group, 0)
            c_x.wait()
            pltpu.async_copy(nrow_v, xcat.at[pl.ds(base, NCH)], s_out).wait()

        return carry

    lax.fori_loop(0, NODE_ITERS, node_iter, 0)


@functools.partial(jax.jit, static_argnames=())
def kernel(x, eattr, ntypes, etypes, ntype_table, etype_table):
    run = pl.kernel(
        _body,
        out_type=(
            jax.ShapeDtypeStruct((N, NODE_W), jnp.float32),
            jax.ShapeDtypeStruct((E, EDGE_W), jnp.float32),
        ),
        mesh=plsc.VectorSubcoreMesh(core_axis_name="c", subcore_axis_name="s"),
        compiler_params=pltpu.CompilerParams(use_tc_tiling_on_sc=False,
                                              needs_layout_passes=False),
        scratch_types=[
            pltpu.VMEM((NUM_NTYPES, NTYPE_DIM), jnp.float32),
            pltpu.VMEM((NUM_ETYPES, ETYPE_DIM), jnp.float32),
            pltpu.VMEM((NCH,), jnp.int32),
            pltpu.VMEM((NCH, NODE_W), jnp.float32),
            pltpu.VMEM((EC,), jnp.int32),
            pltpu.VMEM((EC, EDGE_W), jnp.float32),
            pltpu.SemaphoreType.DMA,
            pltpu.SemaphoreType.DMA,
            pltpu.SemaphoreType.DMA,
        ],
    )
    x_cat, eattr_cat = run(x, eattr, ntypes.astype(jnp.int32),
                           etypes.astype(jnp.int32), ntype_table, etype_table)
    return (x_cat, eattr_cat)


# speedup vs baseline: 3.8974x; 2.3965x over previous
import functools

import jax
import jax.numpy as jnp
from jax import lax
from jax.experimental import pallas as pl
from jax.experimental.pallas import tpu as pltpu
from jax.experimental.pallas import tpu_sc as plsc

N = 10000
E = 320000
D_FEAT = 128
D_EDGE = 16
NTYPE_DIM = 64
ETYPE_DIM = 16
NODE_W = NTYPE_DIM + D_FEAT
EDGE_W = ETYPE_DIM + D_EDGE
NUM_NTYPES = 119
NUM_ETYPES = 22

NC = 2
NS = 16
NW = NC * NS
L = 16

EC = 1280
E_CHUNKS = E // EC
E_ITERS = -(-E_CHUNKS // NW)

NCH = 128
N_FULL = N // NCH
N_TAIL = N - N_FULL * NCH
N_CHUNKS = N_FULL + 1
N_PAD = N_CHUNKS * NCH
N_ITERS = -(-N_CHUNKS // NW)


def _body(x, eattrT, ntypes, etypes, ntab, etab, xcatT, ecatT,
          ntab_v, etab_v, nidx_v, eidx_v, nstage_v, estage_v, xbuf_v,
          s_idx, s_feat, s_out):
    wid = lax.axis_index("s") * NC + lax.axis_index("c")
    iota = lax.broadcasted_iota(jnp.int32, (L,), 0)

    pltpu.sync_copy(ntab, ntab_v)
    pltpu.sync_copy(etab, etab_v)

    def edge_iter(k, carry):
        c = wid + k * NW

        @pl.when(c < E_CHUNKS)
        def _():
            base = pl.multiple_of(c * EC, 128)
            c_idx = pltpu.async_copy(etypes.at[pl.ds(base, EC)], eidx_v, s_idx)
            c_feat = pltpu.async_copy(
                eattrT.at[:, pl.ds(base, EC)],
                estage_v.at[pl.ds(ETYPE_DIM, D_EDGE), :], s_feat)
            c_idx.wait()

            def group(g, carry2):
                ev = eidx_v[pl.ds(g * L, L)]
                for d in range(ETYPE_DIM):
                    dv = jnp.full((L,), d, jnp.int32)
                    vals = plsc.load_gather(etab_v, [ev, dv])
                    estage_v[d, pl.ds(g * L, L)] = vals
                return carry2

            lax.fori_loop(0, EC // L, group, 0)
            c_feat.wait()
            pltpu.async_copy(estage_v, ecatT.at[:, pl.ds(base, EC)], s_out).wait()

        return carry

    lax.fori_loop(0, E_ITERS, edge_iter, 0)

    def do_node_chunk(base, ncols, nidx, nstage, xbuf):
        c_idx = pltpu.async_copy(ntypes.at[pl.ds(base, ncols)],
                                 nidx.at[pl.ds(0, ncols)], s_idx)
        c_x = pltpu.async_copy(x.at[pl.ds(base, ncols)],
                               xbuf.at[pl.ds(0, ncols)], s_feat)
        c_idx.wait()

        def group(g, carry2):
            nv = nidx[pl.ds(g * L, L)]
            for d in range(NTYPE_DIM):
                dv = jnp.full((L,), d, jnp.int32)
                vals = plsc.load_gather(ntab_v, [nv, dv])
                nstage[d, pl.ds(g * L, L)] = vals
            return carry2

        lax.fori_loop(0, ncols // L, group, 0)
        c_x.wait()

        def tgroup(g, carry2):
            colv = iota + g * L
            for f in range(D_FEAT):
                fv = jnp.full((L,), f, jnp.int32)
                vals = plsc.load_gather(xbuf, [colv, fv])
                nstage[NTYPE_DIM + f, pl.ds(g * L, L)] = vals
            return carry2

        lax.fori_loop(0, ncols // L, tgroup, 0)

    def node_iter(k, carry):
        c = wid + k * NW

        @pl.when(c < N_FULL)
        def _():
            base = pl.multiple_of(c * NCH, 128)
            do_node_chunk(base, NCH, nidx_v, nstage_v, xbuf_v)
            pltpu.async_copy(nstage_v, xcatT.at[:, pl.ds(base, NCH)], s_out).wait()

        @pl.when(c == N_FULL)
        def _():
            base = N_FULL * NCH
            do_node_chunk(base, N_TAIL, nidx_v, nstage_v, xbuf_v)
            pltpu.async_copy(nstage_v, xcatT.at[:, pl.ds(base, NCH)], s_out).wait()

        return carry

    lax.fori_loop(0, N_ITERS, node_iter, 0)


@functools.partial(jax.jit, static_argnames=())
def kernel(x, eattr, ntypes, etypes, ntype_table, etype_table):
    run = pl.kernel(
        _body,
        out_type=(
            jax.ShapeDtypeStruct((NODE_W, N_PAD), jnp.float32),
            jax.ShapeDtypeStruct((EDGE_W, E), jnp.float32),
        ),
        mesh=plsc.VectorSubcoreMesh(core_axis_name="c", subcore_axis_name="s"),
        compiler_params=pltpu.CompilerParams(use_tc_tiling_on_sc=True,
                                             needs_layout_passes=False),
        scratch_types=[
            pltpu.VMEM((NUM_NTYPES, NTYPE_DIM), jnp.float32),
            pltpu.VMEM((NUM_ETYPES, ETYPE_DIM), jnp.float32),
            pltpu.VMEM((NCH,), jnp.int32),
            pltpu.VMEM((EC,), jnp.int32),
            pltpu.VMEM((NODE_W, NCH), jnp.float32),
            pltpu.VMEM((EDGE_W, EC), jnp.float32),
            pltpu.VMEM((NCH, D_FEAT), jnp.float32),
            pltpu.SemaphoreType.DMA,
            pltpu.SemaphoreType.DMA,
            pltpu.SemaphoreType.DMA,
        ],
    )
    xcatT, ecatT = run(x, jnp.transpose(eattr), ntypes.astype(jnp.int32),
                       etypes.astype(jnp.int32), ntype_table, etype_table)
    return (jnp.transpose(xcatT)[:N], jnp.transpose(ecatT))


# speedup vs baseline: 3.9793x; 1.0210x over previous
import functools

import jax
import jax.numpy as jnp
from jax import lax
from jax.experimental import pallas as pl
from jax.experimental.pallas import tpu as pltpu
from jax.experimental.pallas import tpu_sc as plsc

N = 10000
E = 320000
D_FEAT = 128
D_EDGE = 16
NTYPE_DIM = 64
ETYPE_DIM = 16
NODE_W = NTYPE_DIM + D_FEAT
EDGE_W = ETYPE_DIM + D_EDGE
NUM_NTYPES = 119
NUM_ETYPES = 22

NC = 2
NS = 16
NW = NC * NS
L = 16

EC = 640
E_CHUNKS = E // EC
EU = E_CHUNKS // NW
E_TAILW = E_CHUNKS - EU * NW
EGROUPS = EC // L

NCH = 128
N_FULL = N // NCH
N_TAIL = N - N_FULL * NCH
N_CHUNKS = N_FULL + 1
N_PAD = N_CHUNKS * NCH
N_ITERS = -(-N_CHUNKS // NW)


def _body(x, eattrT, ntypes, etypes, ntab, etab, xcatT, ecatT,
          ntab_v, etab_v, nidx_v, nstage_v, xbuf_v,
          eidx0, eidx1, est0, est1,
          si0, si1, sf0, sf1, so0, so1):
    wid = lax.axis_index("s") * NC + lax.axis_index("c")
    iota = lax.broadcasted_iota(jnp.int32, (L,), 0)

    pltpu.sync_copy(ntab, ntab_v)
    pltpu.sync_copy(etab, etab_v)

    eidx = (eidx0, eidx1)
    est = (est0, est1)
    s_idx = (si0, si1)
    s_feat = (sf0, sf1)
    s_out = (so0, so1)

    def e_issue_in(k, b):
        base = pl.multiple_of((wid + k * NW) * EC, 128)
        pltpu.async_copy(etypes.at[pl.ds(base, EC)], eidx[b], s_idx[b])
        pltpu.async_copy(eattrT.at[:, pl.ds(base, EC)],
                         est[b].at[pl.ds(ETYPE_DIM, D_EDGE), :], s_feat[b])

    def e_wait_idx(b):
        pltpu.make_async_copy(etypes.at[pl.ds(0, EC)], eidx[b], s_idx[b]).wait()

    def e_wait_feat(b):
        pltpu.make_async_copy(eattrT.at[:, pl.ds(0, EC)],
                              est[b].at[pl.ds(ETYPE_DIM, D_EDGE), :],
                              s_feat[b]).wait()

    def e_wait_out(b):
        pltpu.make_async_copy(est[b], ecatT.at[:, pl.ds(0, EC)], s_out[b]).wait()

    def e_vector(b):
        def group(g, carry):
            ev = eidx[b][pl.ds(g * L, L)]
            for d in range(ETYPE_DIM):
                dv = jnp.full((L,), d, jnp.int32)
                vals = plsc.load_gather(etab_v, [ev, dv])
                est[b][d, pl.ds(g * L, L)] = vals
            return carry

        lax.fori_loop(0, EGROUPS, group, 0)

    def e_issue_out(k, b):
        base = pl.multiple_of((wid + k * NW) * EC, 128)
        pltpu.async_copy(est[b], ecatT.at[:, pl.ds(base, EC)], s_out[b])

    def e_pair(j, carry):
        k0 = j * 2

        @pl.when(k0 > 0)
        def _():
            e_wait_out(1)
        e_issue_in(k0 + 1, 1)
        e_wait_idx(0)
        e_vector(0)
        e_wait_feat(0)
        e_issue_out(k0, 0)

        e_wait_out(0)
        e_issue_in(k0 + 2, 0)
        e_wait_idx(1)
        e_vector(1)
        e_wait_feat(1)
        e_issue_out(k0 + 1, 1)
        return carry

    e_issue_in(0, 0)
    lax.fori_loop(0, (EU - 1) // 2, e_pair, 0)

    e_wait_out(1)

    @pl.when(wid < E_TAILW)
    def _():
        e_issue_in(EU, 1)
    e_wait_idx(0)
    e_vector(0)
    e_wait_feat(0)
    e_issue_out(EU - 1, 0)

    @pl.when(wid < E_TAILW)
    def _():
        e_wait_out(0)
        e_wait_idx(1)
        e_vector(1)
        e_wait_feat(1)
        e_issue_out(EU, 1)
        e_wait_out(1)

    @pl.when(wid >= E_TAILW)
    def _():
        e_wait_out(0)

    def do_node_chunk(base, ncols):
        c_idx = pltpu.async_copy(ntypes.at[pl.ds(base, ncols)],
                                 nidx_v.at[pl.ds(0, ncols)], si0)
        c_x = pltpu.async_copy(x.at[pl.ds(base, ncols)],
                               xbuf_v.at[pl.ds(0, ncols)], sf0)
        c_idx.wait()

        def group(g, carry):
            nv = nidx_v[pl.ds(g * L, L)]
            for d in range(NTYPE_DIM):
                dv = jnp.full((L,), d, jnp.int32)
                vals = plsc.load_gather(ntab_v, [nv, dv])
                nstage_v[d, pl.ds(g * L, L)] = vals
            return carry

        lax.fori_loop(0, ncols // L, group, 0)
        c_x.wait()

        def tgroup(g, carry):
            colv = iota + g * L
            for f in range(D_FEAT):
                fv = jnp.full((L,), f, jnp.int32)
                vals = plsc.load_gather(xbuf_v, [colv, fv])
                nstage_v[NTYPE_DIM + f, pl.ds(g * L, L)] = vals
            return carry

        lax.fori_loop(0, ncols // L, tgroup, 0)

    def node_iter(k, carry):
        c = wid + k * NW

        @pl.when(c < N_FULL)
        def _():
            base = pl.multiple_of(c * NCH, 128)
            do_node_chunk(base, NCH)
            pltpu.async_copy(nstage_v, xcatT.at[:, pl.ds(base, NCH)], so0).wait()

        @pl.when(c == N_FULL)
        def _():
            base = N_FULL * NCH
            do_node_chunk(base, N_TAIL)
            pltpu.async_copy(nstage_v, xcatT.at[:, pl.ds(base, NCH)], so0).wait()

        return carry

    lax.fori_loop(0, N_ITERS, node_iter, 0)


@functools.partial(jax.jit, static_argnames=())
def kernel(x, eattr, ntypes, etypes, ntype_table, etype_table):
    run = pl.kernel(
        _body,
        out_type=(
            jax.ShapeDtypeStruct((NODE_W, N_PAD), jnp.float32),
            jax.ShapeDtypeStruct((EDGE_W, E), jnp.float32),
        ),
        mesh=plsc.VectorSubcoreMesh(core_axis_name="c", subcore_axis_name="s"),
        compiler_params=pltpu.CompilerParams(use_tc_tiling_on_sc=True,
                                             needs_layout_passes=False),
        scratch_types=[
            pltpu.VMEM((NUM_NTYPES, NTYPE_DIM), jnp.float32),
            pltpu.VMEM((NUM_ETYPES, ETYPE_DIM), jnp.float32),
            pltpu.VMEM((NCH,), jnp.int32),
            pltpu.VMEM((NODE_W, NCH), jnp.float32),
            pltpu.VMEM((NCH, D_FEAT), jnp.float32),
            pltpu.VMEM((EC,), jnp.int32),
            pltpu.VMEM((EC,), jnp.int32),
            pltpu.VMEM((EDGE_W, EC), jnp.float32),
            pltpu.VMEM((EDGE_W, EC), jnp.float32),
            pltpu.SemaphoreType.DMA,
            pltpu.SemaphoreType.DMA,
            pltpu.SemaphoreType.DMA,
            pltpu.SemaphoreType.DMA,
            pltpu.SemaphoreType.DMA,
            pltpu.SemaphoreType.DMA,
        ],
    )
    xcatT, ecatT = run(x, jnp.transpose(eattr), ntypes.astype(jnp.int32),
                       etypes.astype(jnp.int32), ntype_table, etype_table)
    return (jnp.transpose(xcatT)[:N], jnp.transpose(ecatT))


# speedup vs baseline: 5.9614x; 1.4981x over previous
import functools

import jax
import jax.numpy as jnp
from jax import lax
from jax.experimental import pallas as pl
from jax.experimental.pallas import tpu as pltpu
from jax.experimental.pallas import tpu_sc as plsc

N = 10000
E = 320000
D_FEAT = 128
D_EDGE = 16
NTYPE_DIM = 64
ETYPE_DIM = 16
NODE_W = NTYPE_DIM + D_FEAT
EDGE_W = ETYPE_DIM + D_EDGE
NUM_NTYPES = 119
NUM_ETYPES = 22

NC = 2
NS = 16
NW = NC * NS
L = 16

EC = 640
E_CHUNKS = E // EC
EU = E_CHUNKS // NW
E_TAILW = E_CHUNKS - EU * NW
EGROUPS = EC // L

NCH = 128
N_FULL = N // NCH
N_TAIL = N - N_FULL * NCH
N_CHUNKS = N_FULL + 1
N_PAD = N_CHUNKS * NCH
N_ITERS = -(-N_CHUNKS // NW)


def _body(x, eattrT, ntypes, etypes, ntab, etab, xcatT, ecatT,
          ntab_v, etab_v, nidx_v, nstage_v, xbuf_v,
          eidx0, eidx1, est0, est1,
          si0, si1, sf0, sf1, so0, so1):
    wid = lax.axis_index("s") * NC + lax.axis_index("c")
    iota = lax.broadcasted_iota(jnp.int32, (L,), 0)

    pltpu.sync_copy(ntab, ntab_v)
    pltpu.sync_copy(etab, etab_v)

    eidx = (eidx0, eidx1)
    est = (est0, est1)
    s_idx = (si0, si1)
    s_feat = (sf0, sf1)
    s_out = (so0, so1)

    def e_issue_in(k, b):
        base = pl.multiple_of((wid + k * NW) * EC, 128)
        pltpu.async_copy(etypes.at[pl.ds(base, EC)], eidx[b], s_idx[b])
        pltpu.async_copy(eattrT.at[:, pl.ds(base, EC)],
                         est[b].at[pl.ds(ETYPE_DIM, D_EDGE), :], s_feat[b])

    def e_wait_idx(b):
        pltpu.make_async_copy(etypes.at[pl.ds(0, EC)], eidx[b], s_idx[b]).wait()

    def e_wait_feat(b):
        pltpu.make_async_copy(eattrT.at[:, pl.ds(0, EC)],
                              est[b].at[pl.ds(ETYPE_DIM, D_EDGE), :],
                              s_feat[b]).wait()

    def e_wait_out(b):
        pltpu.make_async_copy(est[b], ecatT.at[:, pl.ds(0, EC)], s_out[b]).wait()

    def e_vector(b):
        @plsc.parallel_loop(0, EGROUPS, unroll=2)
        def _group(g):
            ev = eidx[b][pl.ds(g * L, L)]
            for d in range(ETYPE_DIM):
                dv = jnp.full((L,), d, jnp.int32)
                vals = plsc.load_gather(etab_v, [ev, dv])
                est[b][d, pl.ds(g * L, L)] = vals

    def e_issue_out(k, b):
        base = pl.multiple_of((wid + k * NW) * EC, 128)
        pltpu.async_copy(est[b], ecatT.at[:, pl.ds(base, EC)], s_out[b])

    def e_pair(j, carry):
        k0 = j * 2

        @pl.when(k0 > 0)
        def _():
            e_wait_out(1)
        e_issue_in(k0 + 1, 1)
        e_wait_idx(0)
        e_vector(0)
        e_wait_feat(0)
        e_issue_out(k0, 0)

        e_wait_out(0)
        e_issue_in(k0 + 2, 0)
        e_wait_idx(1)
        e_vector(1)
        e_wait_feat(1)
        e_issue_out(k0 + 1, 1)
        return carry

    e_issue_in(0, 0)
    lax.fori_loop(0, (EU - 1) // 2, e_pair, 0)

    e_wait_out(1)

    @pl.when(wid < E_TAILW)
    def _():
        e_issue_in(EU, 1)
    e_wait_idx(0)
    e_vector(0)
    e_wait_feat(0)
    e_issue_out(EU - 1, 0)

    @pl.when(wid < E_TAILW)
    def _():
        e_wait_out(0)
        e_wait_idx(1)
        e_vector(1)
        e_wait_feat(1)
        e_issue_out(EU, 1)
        e_wait_out(1)

    @pl.when(wid >= E_TAILW)
    def _():
        e_wait_out(0)

    def do_node_chunk(base, ncols):
        c_idx = pltpu.async_copy(ntypes.at[pl.ds(base, ncols)],
                                 nidx_v.at[pl.ds(0, ncols)], si0)
        c_x = pltpu.async_copy(x.at[pl.ds(base, ncols)],
                               xbuf_v.at[pl.ds(0, ncols)], sf0)
        c_idx.wait()

        u = 2 if ncols // L >= 2 else 1

        @plsc.parallel_loop(0, ncols // L, unroll=u)
        def _group(g):
            nv = nidx_v[pl.ds(g * L, L)]
            for d in range(NTYPE_DIM):
                dv = jnp.full((L,), d, jnp.int32)
                vals = plsc.load_gather(ntab_v, [nv, dv])
                nstage_v[d, pl.ds(g * L, L)] = vals

        c_x.wait()

        @plsc.parallel_loop(0, ncols // L, unroll=u)
        def _tgroup(g):
            colv = iota + g * L
            for f in range(D_FEAT):
                fv = jnp.full((L,), f, jnp.int32)
                vals = plsc.load_gather(xbuf_v, [colv, fv])
                nstage_v[NTYPE_DIM + f, pl.ds(g * L, L)] = vals

    def node_iter(k, carry):
        c = wid + k * NW

        @pl.when(c < N_FULL)
        def _():
            base = pl.multiple_of(c * NCH, 128)
            do_node_chunk(base, NCH)
            pltpu.async_copy(nstage_v, xcatT.at[:, pl.ds(base, NCH)], so0).wait()

        @pl.when(c == N_FULL)
        def _():
            base = N_FULL * NCH
            do_node_chunk(base, N_TAIL)
            pltpu.async_copy(nstage_v, xcatT.at[:, pl.ds(base, NCH)], so0).wait()

        return carry

    lax.fori_loop(0, N_ITERS, node_iter, 0)


@functools.partial(jax.jit, static_argnames=())
def kernel(x, eattr, ntypes, etypes, ntype_table, etype_table):
    run = pl.kernel(
        _body,
        out_type=(
            jax.ShapeDtypeStruct((NODE_W, N_PAD), jnp.float32),
            jax.ShapeDtypeStruct((EDGE_W, E), jnp.float32),
        ),
        mesh=plsc.VectorSubcoreMesh(core_axis_name="c", subcore_axis_name="s"),
        compiler_params=pltpu.CompilerParams(use_tc_tiling_on_sc=True,
                                             needs_layout_passes=False),
        scratch_types=[
            pltpu.VMEM((NUM_NTYPES, NTYPE_DIM), jnp.float32),
            pltpu.VMEM((NUM_ETYPES, ETYPE_DIM), jnp.float32),
            pltpu.VMEM((NCH,), jnp.int32),
            pltpu.VMEM((NODE_W, NCH), jnp.float32),
            pltpu.VMEM((NCH, D_FEAT), jnp.float32),
            pltpu.VMEM((EC,), jnp.int32),
            pltpu.VMEM((EC,), jnp.int32),
            pltpu.VMEM((EDGE_W, EC), jnp.float32),
            pltpu.VMEM((EDGE_W, EC), jnp.float32),
            pltpu.SemaphoreType.DMA,
            pltpu.SemaphoreType.DMA,
            pltpu.SemaphoreType.DMA,
            pltpu.SemaphoreType.DMA,
            pltpu.SemaphoreType.DMA,
            pltpu.SemaphoreType.DMA,
        ],
    )
    xcatT, ecatT = run(x, jnp.transpose(eattr), ntypes.astype(jnp.int32),
                       etypes.astype(jnp.int32), ntype_table, etype_table)
    return (jnp.transpose(xcatT)[:N], jnp.transpose(ecatT))
